# baseline (device time: 87603 ns/iter reference)
import jax
import jax.numpy as jnp
from jax import lax
from jax.experimental import pallas as pl
from jax.experimental.pallas import tpu as pltpu

N_DEV = 4


def kernel(x, w_mat):
    m_per, k = x.shape
    _, n_per = w_mat.shape

    def body(x_ref, w_ref, out_ref, comm_ref, w_bf_ref, send_sems, recv_sems):
        my_pos = lax.axis_index("i")
        left = (my_pos - 1) % N_DEV
        right = (my_pos + 1) % N_DEV

        barrier_sem = pltpu.get_barrier_semaphore()
        for nbr in [left, right]:
            pl.semaphore_signal(
                barrier_sem, inc=1,
                device_id=(nbr,), device_id_type=pl.DeviceIdType.MESH,
            )
        pl.semaphore_wait(barrier_sem, 2)

        w_bf_ref[...] = w_ref[...].astype(jnp.bfloat16)
        comm_ref[0] = x_ref[...].astype(jnp.bfloat16)

        out_ref[pl.ds(my_pos * m_per, m_per), :] = jnp.dot(
            comm_ref[0], w_bf_ref[...], preferred_element_type=jnp.float32
        )

        for h in range(N_DEV - 1):
            rdma = pltpu.make_async_remote_copy(
                src_ref=comm_ref.at[h],
                dst_ref=comm_ref.at[h + 1],
                send_sem=send_sems.at[h],
                recv_sem=recv_sems.at[h],
                device_id=(right,),
                device_id_type=pl.DeviceIdType.MESH,
            )
            rdma.start()
            rdma.wait()
            origin = (my_pos - h - 1) % N_DEV
            out_ref[pl.ds(origin * m_per, m_per), :] = jnp.dot(
                comm_ref[h + 1], w_bf_ref[...], preferred_element_type=jnp.float32
            )

    return pl.pallas_call(
        body,
        out_shape=jax.ShapeDtypeStruct((N_DEV * m_per, n_per), jnp.float32),
        in_specs=[
            pl.BlockSpec(memory_space=pltpu.VMEM),
            pl.BlockSpec(memory_space=pltpu.VMEM),
        ],
        out_specs=pl.BlockSpec(memory_space=pltpu.VMEM),
        scratch_shapes=[
            pltpu.VMEM((N_DEV, m_per, k), jnp.bfloat16),
            pltpu.VMEM((k, n_per), jnp.bfloat16),
            pltpu.SemaphoreType.DMA((N_DEV - 1,)),
            pltpu.SemaphoreType.DMA((N_DEV - 1,)),
        ],
        compiler_params=pltpu.CompilerParams(collective_id=0),
    )(x, w_mat)


# device time: 48082 ns/iter; 1.8220x vs baseline; 1.8220x over previous
import jax
import jax.numpy as jnp
from jax import lax
from jax.experimental import pallas as pl
from jax.experimental.pallas import tpu as pltpu

N_DEV = 4


def kernel(x, w_mat):
    m_per, k = x.shape
    _, n_per = w_mat.shape
    m_half = m_per // 2

    def body(x_ref, w_ref, out_ref, own, buf_l, buf_r, buf_d, w_bf,
             ssems, rsems):
        my_pos = lax.axis_index("i")
        left = (my_pos - 1) % N_DEV
        right = (my_pos + 1) % N_DEV
        org_l = left
        org_r = right
        org_d = (my_pos + 2) % N_DEV

        own[...] = x_ref[...].reshape(2, m_half, k).astype(jnp.bfloat16)
        w_bf[...] = w_ref[...].astype(jnp.bfloat16)

        barrier_sem = pltpu.get_barrier_semaphore()
        for nbr in [left, right]:
            pl.semaphore_signal(
                barrier_sem, inc=1,
                device_id=(nbr,), device_id_type=pl.DeviceIdType.MESH,
            )
        pl.semaphore_wait(barrier_sem, 2)

        send_r = pltpu.make_async_remote_copy(
            src_ref=own, dst_ref=buf_l,
            send_sem=ssems.at[0], recv_sem=rsems.at[0],
            device_id=(right,), device_id_type=pl.DeviceIdType.MESH,
        )
        send_r.start()
        send_l = pltpu.make_async_remote_copy(
            src_ref=own, dst_ref=buf_r,
            send_sem=ssems.at[1], recv_sem=rsems.at[1],
            device_id=(left,), device_id_type=pl.DeviceIdType.MESH,
        )
        send_l.start()

        out_ref[pl.ds(my_pos * m_per, m_per), :] = jnp.dot(
            own[...].reshape(m_per, k), w_bf[...],
            preferred_element_type=jnp.float32,
        )

        recv_l = pltpu.make_async_remote_copy(
            src_ref=own, dst_ref=buf_l,
            send_sem=ssems.at[0], recv_sem=rsems.at[0],
            device_id=(left,), device_id_type=pl.DeviceIdType.MESH,
        )
        recv_l.wait_recv()
        fwd_r = pltpu.make_async_remote_copy(
            src_ref=buf_l.at[1], dst_ref=buf_d.at[1],
            send_sem=ssems.at[2], recv_sem=rsems.at[2],
            device_id=(right,), device_id_type=pl.DeviceIdType.MESH,
        )
        fwd_r.start()

        recv_r = pltpu.make_async_remote_copy(
            src_ref=own, dst_ref=buf_r,
            send_sem=ssems.at[1], recv_sem=rsems.at[1],
            device_id=(right,), device_id_type=pl.DeviceIdType.MESH,
        )
        recv_r.wait_recv()
        fwd_l = pltpu.make_async_remote_copy(
            src_ref=buf_r.at[0], dst_ref=buf_d.at[0],
            send_sem=ssems.at[3], recv_sem=rsems.at[3],
            device_id=(left,), device_id_type=pl.DeviceIdType.MESH,
        )
        fwd_l.start()

        out_ref[pl.ds(org_l * m_per, m_per), :] = jnp.dot(
            buf_l[...].reshape(m_per, k), w_bf[...],
            preferred_element_type=jnp.float32,
        )
        out_ref[pl.ds(org_r * m_per, m_per), :] = jnp.dot(
            buf_r[...].reshape(m_per, k), w_bf[...],
            preferred_element_type=jnp.float32,
        )

        recv_d_bot = pltpu.make_async_remote_copy(
            src_ref=buf_l.at[1], dst_ref=buf_d.at[1],
            send_sem=ssems.at[2], recv_sem=rsems.at[2],
            device_id=(left,), device_id_type=pl.DeviceIdType.MESH,
        )
        recv_d_bot.wait_recv()
        recv_d_top = pltpu.make_async_remote_copy(
            src_ref=buf_r.at[0], dst_ref=buf_d.at[0],
            send_sem=ssems.at[3], recv_sem=rsems.at[3],
            device_id=(right,), device_id_type=pl.DeviceIdType.MESH,
        )
        recv_d_top.wait_recv()

        out_ref[pl.ds(org_d * m_per, m_per), :] = jnp.dot(
            buf_d[...].reshape(m_per, k), w_bf[...],
            preferred_element_type=jnp.float32,
        )

        send_r.wait_send()
        send_l.wait_send()
        fwd_r.wait_send()
        fwd_l.wait_send()

    return pl.pallas_call(
        body,
        out_shape=jax.ShapeDtypeStruct((N_DEV * m_per, n_per), jnp.float32),
        in_specs=[
            pl.BlockSpec(memory_space=pltpu.VMEM),
            pl.BlockSpec(memory_space=pltpu.VMEM),
        ],
        out_specs=pl.BlockSpec(memory_space=pltpu.VMEM),
        scratch_shapes=[
            pltpu.VMEM((2, m_half, k), jnp.bfloat16),
            pltpu.VMEM((2, m_half, k), jnp.bfloat16),
            pltpu.VMEM((2, m_half, k), jnp.bfloat16),
            pltpu.VMEM((2, m_half, k), jnp.bfloat16),
            pltpu.VMEM((k, n_per), jnp.bfloat16),
            pltpu.SemaphoreType.DMA((4,)),
            pltpu.SemaphoreType.DMA((4,)),
        ],
        compiler_params=pltpu.CompilerParams(collective_id=0),
    )(x, w_mat)


# device time: 46522 ns/iter; 1.8830x vs baseline; 1.0335x over previous
import jax
import jax.numpy as jnp
from jax import lax
from jax.experimental import pallas as pl
from jax.experimental.pallas import tpu as pltpu

N_DEV = 4


def kernel(x, w_mat):
    m_per, k = x.shape
    _, n_per = w_mat.shape
    m_half = m_per // 2

    def body(x_ref, w_ref, out_ref, own, buf_l, buf_r, buf_d, w_bf,
             ssems, rsems):
        my_pos = lax.axis_index("i")
        left = (my_pos - 1) % N_DEV
        right = (my_pos + 1) % N_DEV
        org_l = left
        org_r = right
        org_d = (my_pos + 2) % N_DEV

        def rdma(src, dst, i, dev):
            return pltpu.make_async_remote_copy(
                src_ref=src, dst_ref=dst,
                send_sem=ssems.at[i], recv_sem=rsems.at[i],
                device_id=(dev,), device_id_type=pl.DeviceIdType.MESH,
            )

        own[...] = x_ref[...].reshape(2, m_half, k).astype(jnp.bfloat16)

        barrier_sem = pltpu.get_barrier_semaphore()
        for nbr in [left, right]:
            pl.semaphore_signal(
                barrier_sem, inc=1,
                device_id=(nbr,), device_id_type=pl.DeviceIdType.MESH,
            )
        pl.semaphore_wait(barrier_sem, 2)

        s0 = rdma(own.at[1], buf_l.at[1], 0, right)
        s0.start()
        s3 = rdma(own.at[0], buf_r.at[0], 3, left)
        s3.start()
        s1 = rdma(own.at[0], buf_l.at[0], 1, right)
        s1.start()
        s4 = rdma(own.at[1], buf_r.at[1], 4, left)
        s4.start()

        w_bf[...] = w_ref[...].astype(jnp.bfloat16)
        out_ref[pl.ds(my_pos * m_per, m_per), :] = jnp.dot(
            own[...].reshape(m_per, k), w_bf[...],
            preferred_element_type=jnp.float32,
        )

        rdma(own.at[1], buf_l.at[1], 0, left).wait_recv()
        s2 = rdma(buf_l.at[1], buf_d.at[1], 2, right)
        s2.start()
        rdma(own.at[0], buf_r.at[0], 3, right).wait_recv()
        s5 = rdma(buf_r.at[0], buf_d.at[0], 5, left)
        s5.start()

        out_ref[pl.ds(org_l * m_per + m_half, m_half), :] = jnp.dot(
            buf_l[1], w_bf[...], preferred_element_type=jnp.float32,
        )
        out_ref[pl.ds(org_r * m_per, m_half), :] = jnp.dot(
            buf_r[0], w_bf[...], preferred_element_type=jnp.float32,
        )

        rdma(own.at[0], buf_l.at[0], 1, left).wait_recv()
        out_ref[pl.ds(org_l * m_per, m_half), :] = jnp.dot(
            buf_l[0], w_bf[...], preferred_element_type=jnp.float32,
        )
        rdma(own.at[1], buf_r.at[1], 4, right).wait_recv()
        out_ref[pl.ds(org_r * m_per + m_half, m_half), :] = jnp.dot(
            buf_r[1], w_bf[...], preferred_element_type=jnp.float32,
        )

        rdma(buf_l.at[1], buf_d.at[1], 2, left).wait_recv()
        rdma(buf_r.at[0], buf_d.at[0], 5, right).wait_recv()
        out_ref[pl.ds(org_d * m_per, m_per), :] = jnp.dot(
            buf_d[...].reshape(m_per, k), w_bf[...],
            preferred_element_type=jnp.float32,
        )

        for s in (s0, s1, s2, s3, s4, s5):
            s.wait_send()

    return pl.pallas_call(
        body,
        out_shape=jax.ShapeDtypeStruct((N_DEV * m_per, n_per), jnp.float32),
        in_specs=[
            pl.BlockSpec(memory_space=pltpu.VMEM),
            pl.BlockSpec(memory_space=pltpu.VMEM),
        ],
        out_specs=pl.BlockSpec(memory_space=pltpu.VMEM),
        scratch_shapes=[
            pltpu.VMEM((2, m_half, k), jnp.bfloat16),
            pltpu.VMEM((2, m_half, k), jnp.bfloat16),
            pltpu.VMEM((2, m_half, k), jnp.bfloat16),
            pltpu.VMEM((2, m_half, k), jnp.bfloat16),
            pltpu.VMEM((k, n_per), jnp.bfloat16),
            pltpu.SemaphoreType.DMA((6,)),
            pltpu.SemaphoreType.DMA((6,)),
        ],
        compiler_params=pltpu.CompilerParams(collective_id=0),
    )(x, w_mat)
